# Initial kernel scaffold; baseline (speedup 1.0000x reference)
import jax
import jax.numpy as jnp
from jax.experimental import pallas as pl


def kernel(x, emb_table, pos_table, pos_ids):
    raise NotImplementedError("write your pallas kernel here")


# speedup vs baseline: 2.4906x; 2.4906x over previous
import functools

import jax
import jax.numpy as jnp
from jax import lax
from jax.experimental import pallas as pl
from jax.experimental.pallas import tpu as pltpu
from jax.experimental.pallas import tpu_sc as plsc

_VOCAB = 1000
_NC = 2
_NS = 16
_NW = _NC * _NS
_LANES = 16


def _make_sc_embed(B, S, D):
    BW = B // _NW
    NV = D // _LANES

    mesh = plsc.VectorSubcoreMesh(core_axis_name="c", subcore_axis_name="s")

    @functools.partial(
        pl.kernel,
        mesh=mesh,
        out_type=jax.ShapeDtypeStruct((B, S, D), jnp.float32),
        scratch_types=[
            pltpu.VMEM((S, D), jnp.float32),
            pltpu.VMEM((BW,), jnp.int32),
            pltpu.VMEM((BW, D), jnp.float32),
            pltpu.SemaphoreType.DMA,
        ],
    )
    def sc_embed(xgT_hbm, emb_hbm, pos_hbm, out_hbm, pos_v, idx_v, rows_v, sem):
        cid = lax.axis_index("c")
        sid = lax.axis_index("s")
        w = sid * _NC + cid
        b0 = w * BW

        pltpu.sync_copy(pos_hbm, pos_v)

        def body(j, carry):
            pltpu.sync_copy(xgT_hbm.at[j, pl.ds(b0, BW)], idx_v)
            pltpu.async_copy(emb_hbm.at[idx_v], rows_v, sem).wait()
            pv = [pos_v[j, pl.ds(v * _LANES, _LANES)] for v in range(NV)]

            def add_row(i, pv):
                for v in range(NV):
                    sl = pl.ds(v * _LANES, _LANES)
                    rows_v[i, sl] = rows_v[i, sl] + pv[v]
                return pv

            lax.fori_loop(0, BW, add_row, pv, unroll=2)
            pltpu.sync_copy(rows_v, out_hbm.at[pl.ds(b0, BW), j])
            return carry

        lax.fori_loop(0, S, body, 0)

    return sc_embed


def kernel(x, emb_table, pos_table, pos_ids):
    B, S = x.shape
    Sg = S + 1
    D = emb_table.shape[1]
    xgT = jnp.concatenate(
        [jnp.full((1, B), _VOCAB, x.dtype), x.T.astype(jnp.int32)], axis=0
    )
    pos_eff = jnp.take(pos_table, pos_ids[0], axis=0).astype(jnp.float32)
    out = _make_sc_embed(B, Sg, D)(xgT, emb_table, pos_eff)
    return out
